# Initial kernel scaffold; baseline (speedup 1.0000x reference)
import jax
import jax.numpy as jnp
from jax.experimental import pallas as pl


def kernel(inp, table):
    raise NotImplementedError("write your pallas kernel here")


# speedup vs baseline: 1.8445x; 1.8445x over previous
import functools

import jax
import jax.numpy as jnp
from jax import lax
from jax.experimental import pallas as pl
from jax.experimental.pallas import tpu as pltpu
from jax.experimental.pallas import tpu_sc as plsc

D_EMBED = 64
NUM_CORES = 2
NUM_SUBCORES = 16
NUM_WORKERS = NUM_CORES * NUM_SUBCORES
CHUNK = 800


@functools.partial(jax.jit, static_argnums=(2,))
def _gather(idx_flat, table, B):
    b_per_w = B // NUM_WORKERS
    n_chunks = b_per_w // CHUNK
    mesh = plsc.VectorSubcoreMesh(core_axis_name="c", subcore_axis_name="s")

    @functools.partial(
        pl.kernel,
        mesh=mesh,
        compiler_params=pltpu.CompilerParams(use_tc_tiling_on_sc=False),
        out_type=jax.ShapeDtypeStruct((B, D_EMBED), jnp.float32),
        scratch_types=[
            pltpu.VMEM((CHUNK,), jnp.int32),
            pltpu.VMEM((CHUNK, D_EMBED), jnp.float32),
            pltpu.SemaphoreType.DMA,
        ],
    )
    def k(idx_hbm, table_hbm, out_hbm, idx_v, rows_v, sem):
        wid = lax.axis_index("s") * NUM_CORES + lax.axis_index("c")
        base = wid * b_per_w

        def body(i, _):
            off = base + i * CHUNK
            pltpu.sync_copy(idx_hbm.at[pl.ds(off, CHUNK)], idx_v)
            pltpu.async_copy(table_hbm.at[idx_v], rows_v, sem).wait()
            pltpu.sync_copy(rows_v, out_hbm.at[pl.ds(off, CHUNK)])
            return 0

        lax.fori_loop(0, n_chunks, body, 0)

    return k(idx_flat, table)


def kernel(inp, table):
    batch, hist = inp.shape
    B = batch * hist
    out = _gather(inp.reshape(B), table, B)
    return out.reshape(batch, hist, D_EMBED)


# speedup vs baseline: 1.8687x; 1.0132x over previous
import functools

import jax
import jax.numpy as jnp
from jax import lax
from jax.experimental import pallas as pl
from jax.experimental.pallas import tpu as pltpu
from jax.experimental.pallas import tpu_sc as plsc

D_EMBED = 64
NUM_CORES = 2
NUM_SUBCORES = 16
NUM_WORKERS = NUM_CORES * NUM_SUBCORES
CHUNK = 800


@functools.partial(jax.jit, static_argnums=(2,))
def _gather(idx_flat, table, B):
    b_per_w = B // NUM_WORKERS
    n_chunks = b_per_w // CHUNK
    mesh = plsc.VectorSubcoreMesh(core_axis_name="c", subcore_axis_name="s")

    @functools.partial(
        pl.kernel,
        mesh=mesh,
        compiler_params=pltpu.CompilerParams(use_tc_tiling_on_sc=False),
        out_type=jax.ShapeDtypeStruct((B, D_EMBED), jnp.float32),
        scratch_types=[
            pltpu.VMEM((CHUNK,), jnp.int32),
            pltpu.VMEM((CHUNK,), jnp.int32),
            pltpu.VMEM((CHUNK, D_EMBED), jnp.float32),
            pltpu.VMEM((CHUNK, D_EMBED), jnp.float32),
            pltpu.SemaphoreType.DMA,
            pltpu.SemaphoreType.DMA,
            pltpu.SemaphoreType.DMA,
            pltpu.SemaphoreType.DMA,
            pltpu.SemaphoreType.DMA,
            pltpu.SemaphoreType.DMA,
        ],
    )
    def k(idx_hbm, table_hbm, out_hbm, i0, i1, r0, r1, si0, si1, sg0, sg1, ss0, ss1):
        wid = lax.axis_index("s") * NUM_CORES + lax.axis_index("c")
        base = wid * b_per_w
        idx_v = (i0, i1)
        rows_v = (r0, r1)
        si = (si0, si1)
        sg = (sg0, sg1)
        ss = (ss0, ss1)

        def idx_copy(i, b):
            return pltpu.make_async_copy(
                idx_hbm.at[pl.ds(base + i * CHUNK, CHUNK)], idx_v[b], si[b]
            )

        def store_copy(i, b):
            return pltpu.make_async_copy(
                rows_v[b], out_hbm.at[pl.ds(base + i * CHUNK, CHUNK)], ss[b]
            )

        for b in range(2):
            idx_copy(b, b).start()

        for i in range(n_chunks):
            b = i % 2
            if i >= 2:
                store_copy(i - 2, b).wait()
            idx_copy(i, b).wait()
            pltpu.async_copy(table_hbm.at[idx_v[b]], rows_v[b], sg[b]).wait()
            if i + 2 < n_chunks:
                idx_copy(i + 2, b).start()
            store_copy(i, b).start()

        for b in range(2):
            store_copy(n_chunks - 2 + b, b).wait()

    return k(idx_flat, table)


def kernel(inp, table):
    batch, hist = inp.shape
    B = batch * hist
    out = _gather(inp.reshape(B), table, B)
    return out.reshape(batch, hist, D_EMBED)


# speedup vs baseline: 1.8709x; 1.0012x over previous
import functools

import jax
import jax.numpy as jnp
from jax import lax
from jax.experimental import pallas as pl
from jax.experimental.pallas import tpu as pltpu
from jax.experimental.pallas import tpu_sc as plsc

D_EMBED = 64
NUM_CORES = 2
NUM_SUBCORES = 16
NUM_WORKERS = NUM_CORES * NUM_SUBCORES
CHUNK = 400
NBUF = 4
SKEW = 2


@functools.partial(jax.jit, static_argnums=(2,))
def _gather(idx_flat, table, B):
    b_per_w = B // NUM_WORKERS
    n_chunks = b_per_w // CHUNK
    mesh = plsc.VectorSubcoreMesh(core_axis_name="c", subcore_axis_name="s")

    @functools.partial(
        pl.kernel,
        mesh=mesh,
        compiler_params=pltpu.CompilerParams(use_tc_tiling_on_sc=False),
        out_type=jax.ShapeDtypeStruct((B, D_EMBED), jnp.float32),
        scratch_types=(
            [pltpu.VMEM((CHUNK,), jnp.int32) for _ in range(NBUF)]
            + [pltpu.VMEM((CHUNK, D_EMBED), jnp.float32) for _ in range(NBUF)]
            + [pltpu.SemaphoreType.DMA for _ in range(3 * NBUF)]
        ),
    )
    def k(idx_hbm, table_hbm, out_hbm, *bufs):
        idx_v = bufs[:NBUF]
        rows_v = bufs[NBUF : 2 * NBUF]
        si = bufs[2 * NBUF : 3 * NBUF]
        sg = bufs[3 * NBUF : 4 * NBUF]
        ss = bufs[4 * NBUF : 5 * NBUF]
        wid = lax.axis_index("s") * NUM_CORES + lax.axis_index("c")
        base = wid * b_per_w

        def idx_copy(i):
            b = i % NBUF
            return pltpu.make_async_copy(
                idx_hbm.at[pl.ds(base + i * CHUNK, CHUNK)], idx_v[b], si[b]
            )

        def gather_copy(i):
            b = i % NBUF
            return pltpu.make_async_copy(table_hbm.at[idx_v[b]], rows_v[b], sg[b])

        def store_copy(i):
            b = i % NBUF
            return pltpu.make_async_copy(
                rows_v[b], out_hbm.at[pl.ds(base + i * CHUNK, CHUNK)], ss[b]
            )

        for j in range(NBUF):
            idx_copy(j).start()

        for i in range(n_chunks):
            if i >= NBUF:
                store_copy(i - NBUF).wait()
            idx_copy(i).wait()
            gather_copy(i).start()
            if i >= SKEW:
                gather_copy(i - SKEW).wait()
                store_copy(i - SKEW).start()
                nxt = i - SKEW + NBUF
                if nxt < n_chunks:
                    idx_copy(nxt).start()

        for i in range(n_chunks - SKEW, n_chunks):
            gather_copy(i).wait()
            store_copy(i).start()
        for i in range(n_chunks - NBUF, n_chunks):
            store_copy(i).wait()

    return k(idx_flat, table)


def kernel(inp, table):
    batch, hist = inp.shape
    B = batch * hist
    out = _gather(inp.reshape(B), table, B)
    return out.reshape(batch, hist, D_EMBED)


# speedup vs baseline: 1.8740x; 1.0016x over previous
import functools

import jax
import jax.numpy as jnp
from jax import lax
from jax.experimental import pallas as pl
from jax.experimental.pallas import tpu as pltpu
from jax.experimental.pallas import tpu_sc as plsc

D_EMBED = 64
NUM_CORES = 2
NUM_SUBCORES = 16
NUM_WORKERS = NUM_CORES * NUM_SUBCORES
NB = 8
NBUF = 4
SKEW = 2


@functools.partial(jax.jit, static_argnums=(2, 3))
def _gather(inp, table, batch, hist):
    rows_per_w = batch // NUM_WORKERS
    n_chunks = rows_per_w // NB
    n_groups = n_chunks // NBUF
    mesh = plsc.VectorSubcoreMesh(core_axis_name="c", subcore_axis_name="s")

    @functools.partial(
        pl.kernel,
        mesh=mesh,
        compiler_params=pltpu.CompilerParams(use_tc_tiling_on_sc=False),
        out_type=jax.ShapeDtypeStruct((batch, hist, D_EMBED), jnp.float32),
        scratch_types=(
            [pltpu.VMEM((NB, hist), jnp.int32) for _ in range(NBUF)]
            + [pltpu.VMEM((NB, hist, D_EMBED), jnp.float32) for _ in range(NBUF)]
            + [pltpu.SemaphoreType.DMA for _ in range(3 * NBUF)]
        ),
    )
    def k(idx_hbm, table_hbm, out_hbm, *bufs):
        idx_v = bufs[:NBUF]
        rows_v = bufs[NBUF : 2 * NBUF]
        si = bufs[2 * NBUF : 3 * NBUF]
        sg = bufs[3 * NBUF : 4 * NBUF]
        ss = bufs[4 * NBUF : 5 * NBUF]
        wid = lax.axis_index("s") * NUM_CORES + lax.axis_index("c")
        base = wid * rows_per_w

        def idx_copy(i, b):
            return pltpu.make_async_copy(
                idx_hbm.at[pl.ds(base + i * NB, NB)], idx_v[b], si[b]
            )

        def gather_copies(i, b):
            return [
                pltpu.make_async_copy(
                    table_hbm.at[idx_v[b].at[j]], rows_v[b].at[j], sg[b]
                )
                for j in range(NB)
            ]

        def store_copy(i, b):
            return pltpu.make_async_copy(
                rows_v[b], out_hbm.at[pl.ds(base + i * NB, NB)], ss[b]
            )

        def step(i, b, *, wait_store, prefetch, drain):
            if wait_store:
                store_copy(i - NBUF, b).wait()
            idx_copy(i, b).wait()
            for c in gather_copies(i, b):
                c.start()
            if drain:
                pb = (b - SKEW) % NBUF
                for c in gather_copies(i - SKEW, pb):
                    c.wait()
                store_copy(i - SKEW, pb).start()
                if prefetch:
                    idx_copy(i - SKEW + NBUF, pb).start()

        for b in range(NBUF):
            idx_copy(b, b).start()

        for b in range(NBUF):
            step(b, b, wait_store=False, prefetch=True, drain=(b >= SKEW))

        def body(g, _):
            i0 = g * NBUF
            for b in range(NBUF):
                step(i0 + b, b, wait_store=True, prefetch=True, drain=True)
            return 0

        lax.fori_loop(1, n_groups - 1, body, 0)

        i0 = (n_groups - 1) * NBUF
        for b in range(NBUF):
            step(i0 + b, b, wait_store=True, drain=True,
                 prefetch=(i0 + b - SKEW + NBUF < n_chunks))

        for i in range(n_chunks - SKEW, n_chunks):
            b = i % NBUF
            for c in gather_copies(i, b):
                c.wait()
            store_copy(i, b).start()
        for i in range(n_chunks - NBUF, n_chunks):
            store_copy(i, i % NBUF).wait()

    return k(inp, table)


def kernel(inp, table):
    batch, hist = inp.shape
    return _gather(inp, table, batch, hist)
